# Initial kernel scaffold; baseline (speedup 1.0000x reference)
import jax
import jax.numpy as jnp
from jax.experimental import pallas as pl


def kernel(x_vis, x_sym, edge_index, edge_attr, params):
    raise NotImplementedError("write your pallas kernel here")


# speedup vs baseline: 20.0220x; 20.0220x over previous
import functools

import jax
import jax.numpy as jnp
from jax import lax
from jax.experimental import pallas as pl
from jax.experimental.pallas import tpu as pltpu
from jax.experimental.pallas import tpu_sc as plsc

N = 50000
E = 800000
H = 4
D = 32

NC = 2
NS = 16
NW = NC * NS

E_PAD = 802816
NP2 = 50176

CH_A = 2048
NCHUNK_A = E_PAD // CH_A
BASE_A = NCHUNK_A // NW
EXTRA_A = NCHUNK_A - BASE_A * NW

CH_B = 512
NCH_B = E_PAD // CH_B // NW

PHASES = 8
PROWS = N // PHASES
TRASH = 64
ROWS = PROWS + TRASH
NCH_ALL = E_PAD // CH_B // NS

_mesh = plsc.VectorSubcoreMesh(core_axis_name="c", subcore_axis_name="s")
_sc_params = pltpu.CompilerParams(needs_layout_passes=False)


def _stats_body(src1, dst1, ea0h, ea1h, s0, s1, s2, s3, d0, d1, d2, d3,
                conv, znp,
                tmp_out, ex_out, den_out,
                tab_b, denacc, idx_b, ea0_b, ea1_b, t_b, ex_b, con_b):
  cid = lax.axis_index("c")
  sid = lax.axis_index("s")
  wid = sid * NC + cid
  nch = BASE_A + jnp.where(wid < EXTRA_A, 1, 0)
  stabs = [s0, s1, s2, s3]
  dtabs = [d0, d1, d2, d3]

  for h in range(H):
    pltpu.sync_copy(stabs[h], tab_b)
    pltpu.sync_copy(conv.at[h], con_b)

    def sub1(j, _):
      ci = j * NW + wid
      base = pl.multiple_of(ci * CH_A, CH_A)
      pltpu.sync_copy(src1.at[pl.ds(base, CH_A)], idx_b)
      pltpu.sync_copy(ea0h.at[pl.ds(base, CH_A)], ea0_b)
      pltpu.sync_copy(ea1h.at[pl.ds(base, CH_A)], ea1_b)
      qa0 = con_b[0]
      qa1 = con_b[1]

      def vec(k, _):
        sl = pl.ds(k * 16, 16)
        s16 = plsc.load_gather(tab_b, [idx_b[sl]])
        t_b[sl] = s16 + qa0 * ea0_b[sl] + qa1 * ea1_b[sl]
        return 0

      lax.fori_loop(0, CH_A // 16, vec, 0)
      pltpu.sync_copy(t_b, tmp_out.at[pl.ds(h * E_PAD + base, CH_A)])
      return 0

    lax.fori_loop(0, nch, sub1, 0)

    pltpu.sync_copy(dtabs[h], tab_b)
    pltpu.sync_copy(znp, denacc)

    def sub2(j, _):
      ci = j * NW + wid
      base = pl.multiple_of(ci * CH_A, CH_A)
      pltpu.sync_copy(dst1.at[pl.ds(base, CH_A)], idx_b)
      pltpu.sync_copy(tmp_out.at[pl.ds(h * E_PAD + base, CH_A)], t_b)
      shf = con_b[2]

      def vec(k, _):
        sl = pl.ds(k * 16, 16)
        dst16 = idx_b[sl]
        t = t_b[sl] + plsc.load_gather(tab_b, [dst16])
        t = jnp.maximum(t, 0.2 * t)
        ex16 = jnp.exp(t - shf)
        geid = base + k * 16 + lax.iota(jnp.int32, 16)
        ex16 = jnp.where(geid < E, ex16, 0.0)
        ex_b[sl] = ex16
        plsc.addupdate_scatter(denacc, [dst16], ex16)
        return 0

      lax.fori_loop(0, CH_A // 16, vec, 0)
      pltpu.sync_copy(ex_b, ex_out.at[pl.ds(h * E_PAD + base, CH_A)])
      return 0

    lax.fori_loop(0, nch, sub2, 0)
    pltpu.sync_copy(denacc, den_out.at[wid, h])


def _sc_stats(src1, dst1, ea0, ea1, sL, dL, conv, znp):
  kern = pl.kernel(
      _stats_body,
      mesh=_mesh,
      compiler_params=_sc_params,
      out_type=[
          jax.ShapeDtypeStruct((H * E_PAD,), jnp.float32),
          jax.ShapeDtypeStruct((H * E_PAD,), jnp.float32),
          jax.ShapeDtypeStruct((NW, H, NP2), jnp.float32),
      ],
      scratch_types=[
          pltpu.VMEM((N,), jnp.float32),
          pltpu.VMEM((NP2,), jnp.float32),
          pltpu.VMEM((CH_A,), jnp.int32),
          pltpu.VMEM((CH_A,), jnp.float32),
          pltpu.VMEM((CH_A,), jnp.float32),
          pltpu.VMEM((CH_A,), jnp.float32),
          pltpu.VMEM((CH_A,), jnp.float32),
          pltpu.VMEM((3, 16), jnp.float32),
      ],
  )
  return kern(src1, dst1, ea0, ea1, *sL, *dL, conv, znp)


def _msgs_body(src1, ea0h, ea1h, exT, hfull, wec,
               msg_out,
               hrow_b, src_b, ea0_b, ea1_b, exq_b, wec_b, sem):
  cid = lax.axis_index("c")
  sid = lax.axis_index("s")
  wid = sid * NC + cid

  pltpu.sync_copy(wec, wec_b)
  we0 = [wec_b[0, pl.ds(j * 16, 16)] for j in range(8)]
  we1 = [wec_b[1, pl.ds(j * 16, 16)] for j in range(8)]

  def chunk(j, _):
    ci = j * NW + wid
    base = pl.multiple_of(ci * CH_B, CH_B)
    pltpu.sync_copy(src1.at[pl.ds(base, CH_B)], src_b)
    pltpu.sync_copy(ea0h.at[pl.ds(base, CH_B)], ea0_b)
    pltpu.sync_copy(ea1h.at[pl.ds(base, CH_B)], ea1_b)
    for h in range(H):
      pltpu.sync_copy(exT.at[pl.ds(h * E_PAD + base, CH_B)],
                      exq_b.at[pl.ds(h * CH_B, CH_B)])
    pltpu.async_copy(hfull.at[src_b], hrow_b, sem).wait()

    def edge(e, _):
      mf = jnp.where(jnp.full((16,), 1, jnp.int32) * (base + e) < E, 1.0, 0.0)
      ea0s = plsc.load_gather(ea0_b, [jnp.full((16,), e, jnp.int32)])
      ea1s = plsc.load_gather(ea1_b, [jnp.full((16,), e, jnp.int32)])
      for h in range(H):
        exs = plsc.load_gather(
            exq_b, [jnp.full((16,), h * CH_B, jnp.int32) + e])
        a = exs * mf
        b0 = a * ea0s
        b1 = a * ea1s
        for jj in range(2):
          j = h * 2 + jj
          sl = pl.ds(j * 16, 16)
          v = hrow_b[e, sl]
          hrow_b[e, sl] = a * v + b0 * we0[j] + b1 * we1[j]
      return 0

    lax.fori_loop(0, CH_B, edge, 0)
    pltpu.sync_copy(hrow_b, msg_out.at[pl.ds(base, CH_B), :])
    return 0

  lax.fori_loop(0, NCH_B, chunk, 0)


def _sc_msgs(src1, ea0, ea1, exT, hfull, wec):
  kern = pl.kernel(
      _msgs_body,
      mesh=_mesh,
      compiler_params=_sc_params,
      out_type=jax.ShapeDtypeStruct((E_PAD, 128), jnp.float32),
      scratch_types=[
          pltpu.VMEM((CH_B, 128), jnp.float32),
          pltpu.VMEM((CH_B,), jnp.int32),
          pltpu.VMEM((CH_B,), jnp.float32),
          pltpu.VMEM((CH_B,), jnp.float32),
          pltpu.VMEM((H * CH_B,), jnp.float32),
          pltpu.VMEM((2, 128), jnp.float32),
          pltpu.SemaphoreType.DMA,
      ],
  )
  return kern(src1, ea0, ea1, exT, hfull, wec)


def _scatter_body(dst1, msg, zacc,
                  acc_out,
                  msg_b, dst_b, ridx_b, acc_sh):
  cid = lax.axis_index("c")
  sid = lax.axis_index("s")

  for p in range(PHASES // NC):
    @pl.when(sid == 0)
    def _zero():
      pltpu.sync_copy(zacc, acc_sh)

    plsc.subcore_barrier()
    lo = (cid * (PHASES // NC) + p) * PROWS

    def chunk(j, _):
      ci = j * NS + sid
      base = pl.multiple_of(ci * CH_B, CH_B)
      pltpu.sync_copy(dst1.at[pl.ds(base, CH_B)], dst_b)
      pltpu.sync_copy(msg.at[pl.ds(base, CH_B), :], msg_b)

      def vec(k, _):
        i = k // 8
        c0 = (k % 8) * 16
        dst16 = dst_b[pl.ds(k * 16, 16)]
        inr = jnp.logical_and(dst16 >= lo, dst16 < lo + PROWS)
        route = jnp.where(inr, dst16 - lo, PROWS + jnp.remainder(dst16, TRASH))
        ridx_b[i, pl.ds(c0, 16)] = route
        return 0

      lax.fori_loop(0, CH_B // 16, vec, 0)

      for i in range(CH_B // 128):
        pltpu.sync_copy(msg_b.at[pl.ds(i * 128, 128), :],
                        acc_sh.at[ridx_b.at[i]], add=True)
      return 0

    lax.fori_loop(0, NCH_ALL, chunk, 0)
    plsc.subcore_barrier()

    @pl.when(sid == 0)
    def _copy_out():
      pltpu.sync_copy(acc_sh, acc_out.at[cid * (PHASES // NC) + p])


def _sc_scatter(dst1, msg, zacc):
  kern = pl.kernel(
      _scatter_body,
      mesh=_mesh,
      compiler_params=_sc_params,
      out_type=jax.ShapeDtypeStruct((PHASES, ROWS, 128), jnp.float32),
      scratch_types=[
          pltpu.VMEM((CH_B, 128), jnp.float32),
          pltpu.VMEM((CH_B,), jnp.int32),
          pltpu.VMEM((CH_B // 128, 128), jnp.int32),
          pltpu.VMEM_SHARED((ROWS, 128), jnp.float32),
      ],
  )
  return kern(dst1, msg, zacc)


_BM = 2000
_GRID = N // _BM


def _mm_body(x_ref, w_ref, b_ref, o_ref, *, act):
  y = jnp.dot(x_ref[...], w_ref[...], preferred_element_type=jnp.float32)
  y = y + b_ref[0:1, :]
  if act == "relu":
    y = jnp.maximum(y, 0.0)
  o_ref[...] = y


def _mm(x, w, b, act=None):
  m, k = x.shape
  nc = w.shape[1]
  assert m == N
  return pl.pallas_call(
      functools.partial(_mm_body, act=act),
      grid=(_GRID,),
      in_specs=[
          pl.BlockSpec((_BM, k), lambda i: (i, 0)),
          pl.BlockSpec((k, nc), lambda i: (0, 0)),
          pl.BlockSpec((8, nc), lambda i: (0, 0)),
      ],
      out_specs=pl.BlockSpec((_BM, nc), lambda i: (i, 0)),
      out_shape=jax.ShapeDtypeStruct((m, nc), jnp.float32),
  )(x, w, jnp.broadcast_to(b, (8, nc)))


def _div_body(a_ref, b_ref, o_ref):
  o_ref[...] = a_ref[...] / (b_ref[...] + 1e-16)


def _div(a, b):
  nc = a.shape[1]
  return pl.pallas_call(
      _div_body,
      grid=(_GRID,),
      in_specs=[
          pl.BlockSpec((_BM, nc), lambda i: (i, 0)),
          pl.BlockSpec((_BM, nc), lambda i: (i, 0)),
      ],
      out_specs=pl.BlockSpec((_BM, nc), lambda i: (i, 0)),
      out_shape=jax.ShapeDtypeStruct((N, nc), jnp.float32),
  )(a, b)


def _gate_body(g1_ref, g2_ref, b_ref, agg_ref, res_ref, o_ref):
  gate = jax.nn.sigmoid(g1_ref[...] + g2_ref[...] + b_ref[0:1, :])
  out = gate * agg_ref[...] + (1.0 - gate) * res_ref[...]
  o_ref[...] = jnp.where(out > 0, out, jnp.exp(jnp.minimum(out, 0.0)) - 1.0)


def _gate_fuse(g1, g2, bg, agg, res):
  nc = g1.shape[1]
  return pl.pallas_call(
      _gate_body,
      grid=(_GRID,),
      in_specs=[
          pl.BlockSpec((_BM, nc), lambda i: (i, 0)),
          pl.BlockSpec((_BM, nc), lambda i: (i, 0)),
          pl.BlockSpec((8, nc), lambda i: (0, 0)),
          pl.BlockSpec((_BM, nc), lambda i: (i, 0)),
          pl.BlockSpec((_BM, nc), lambda i: (i, 0)),
      ],
      out_specs=pl.BlockSpec((_BM, nc), lambda i: (i, 0)),
      out_shape=jax.ShapeDtypeStruct((N, nc), jnp.float32),
  )(g1, g2, jnp.broadcast_to(bg, (8, nc)), agg, res)


def _gat_layer(x, src1, dst1, ea0, ea1, eamax, p, znp, zacc):
  in_dim = x.shape[1]
  W = p["W"]
  Ws = (W.reshape(in_dim, H, D) * p["a_src"][None]).sum(-1)
  Wd = (W.reshape(in_dim, H, D) * p["a_dst"][None]).sum(-1)
  qa = (p["We"].reshape(2, H, D) * p["a_e"][None]).sum(-1)

  cat = [W, Ws, Wd, p["Wg"]]
  if "Wr" in p:
    cat.append(p["Wr"])
  Wcat = jnp.concatenate(cat, axis=1)
  big = _mm(x, Wcat, jnp.zeros((Wcat.shape[1],), jnp.float32))
  h = big[:, :128]
  s = big[:, 128:128 + H]
  d = big[:, 128 + H:128 + 2 * H]
  g1 = big[:, 136:264]
  res = big[:, 264:392] if "Wr" in p else x

  sL = [s[:, i] for i in range(H)]
  dL = [d[:, i] for i in range(H)]
  cbound = jnp.abs(qa[0]) * eamax[0] + jnp.abs(qa[1]) * eamax[1]
  tmax = jnp.max(s, axis=0) + jnp.max(d, axis=0) + cbound
  shift = jnp.maximum(tmax, 0.2 * tmax)
  conv = jnp.stack([
      jnp.broadcast_to(qa[0][:, None], (H, 16)),
      jnp.broadcast_to(qa[1][:, None], (H, 16)),
      jnp.broadcast_to(shift[:, None], (H, 16)),
  ], axis=1)

  _, exT, denP = _sc_stats(src1, dst1, ea0, ea1, sL, dL, conv, znp)
  den = denP.sum(axis=0)

  msg = _sc_msgs(src1, ea0, ea1, exT, h, p["We"])
  accP = _sc_scatter(dst1, msg, zacc)

  agg_raw = accP[:, :PROWS, :].reshape(N, 128)
  denx = jnp.repeat(den[:, :N].T, D, axis=1)
  agg = _div(agg_raw, denx)

  g2 = _mm(agg, p["Ug"], jnp.zeros((128,), jnp.float32))
  return _gate_fuse(g1, g2, p["bg"], agg, res)


def kernel(x_vis, x_sym, edge_index, edge_attr, params):
  npad = E_PAD - E
  padidx = (jnp.arange(npad, dtype=jnp.int32) * 16) % N
  src1 = jnp.concatenate([edge_index[0], padidx])
  dst1 = jnp.concatenate([edge_index[1], padidx])
  zpad = jnp.zeros((npad,), jnp.float32)
  ea0 = jnp.concatenate([edge_attr[:, 0], zpad])
  ea1 = jnp.concatenate([edge_attr[:, 1], zpad])
  eamax = jnp.max(jnp.abs(edge_attr), axis=0)

  znp = jnp.zeros((NP2,), jnp.float32)
  zacc = jnp.zeros((ROWS, 128), jnp.float32)

  v_vis = _mm(x_vis, params["W_uv1"], params["b_uv1"], act="relu")
  v_vis = _mm(v_vis, params["W_uv2"], params["b_uv2"])
  v_sym = _mm(x_sym, params["W_sy1"], params["b_sy1"], act="relu")
  v_sym = _mm(v_sym, params["W_sy2"], params["b_sy2"])
  x = jnp.concatenate([v_vis, v_sym], axis=1)

  x = _gat_layer(x, src1, dst1, ea0, ea1, eamax, params["gnn1"], znp, zacc)
  x = _gat_layer(x, src1, dst1, ea0, ea1, eamax, params["gnn2"], znp, zacc)

  hcls = _mm(x, params["Wc1"], params["bc1"], act="relu")
  return _mm(hcls, params["Wc2"], params["bc2"])
